# baseline (device time: 115924 ns/iter reference)
import jax
import jax.numpy as jnp
from jax import lax
from jax.experimental import pallas as pl
from jax.experimental.pallas import tpu as pltpu


def kernel(x, W, labels):
    T, D = x.shape
    _, V = W.shape
    BV = 2048
    nblk = V // BV

    def kernel_body(x_ref, w_ref, l_ref, out_ref, lg_ref, acc_ref, recv_ref,
                    send_sem, recv_sem):
        j = pl.program_id(0)
        my_x = lax.axis_index("x")
        my_y = lax.axis_index("y")
        my_z = lax.axis_index("z")
        cols = lax.broadcasted_iota(jnp.int32, (T, BV), 1)

        def stats(lg, blk):
            bs = jnp.sum(jnp.exp(lg), axis=1)
            loc = l_ref[:] - (my_x * V + blk * BV)
            lab = jnp.sum(jnp.where(cols == loc[:, None], lg, 0.0), axis=1)
            return bs, lab

        lg_ref[j % 2] = jnp.dot(x_ref[:, :], w_ref[:, :],
                                preferred_element_type=jnp.float32)

        bs, lab = stats(lg_ref[(j - 1) % 2], j - 1)
        first = j == 0
        acc_ref[0, :] = (jnp.where(first, 0.0, acc_ref[0, :])
                         + jnp.where(first, 0.0, bs))
        acc_ref[1, :] = (jnp.where(first, 0.0, acc_ref[1, :])
                         + jnp.where(first, 0.0, lab))

        @pl.when(j == nblk - 1)
        def _():
            bs_l, lab_l = stats(lg_ref[(nblk - 1) % 2], nblk - 1)
            acc_ref[0, :] = acc_ref[0, :] + bs_l
            acc_ref[1, :] = acc_ref[1, :] + lab_l

            partner = (1 - my_x, my_y, my_z)
            barrier = pltpu.get_barrier_semaphore()
            pl.semaphore_signal(barrier, inc=1, device_id=partner,
                                device_id_type=pl.DeviceIdType.MESH)
            pl.semaphore_wait(barrier, 1)

            rdma = pltpu.make_async_remote_copy(
                src_ref=acc_ref,
                dst_ref=recv_ref,
                send_sem=send_sem,
                recv_sem=recv_sem,
                device_id=partner,
                device_id_type=pl.DeviceIdType.MESH,
            )
            rdma.start()
            rdma.wait()

            s = acc_ref[0, :] + recv_ref[0, :]
            g = acc_ref[1, :] + recv_ref[1, :]
            out_ref[:] = jnp.log(s) - g

    return pl.pallas_call(
        kernel_body,
        grid=(nblk,),
        out_shape=jax.ShapeDtypeStruct((T,), jnp.float32),
        in_specs=[
            pl.BlockSpec((T, D), lambda j: (0, 0)),
            pl.BlockSpec((D, BV), lambda j: (0, j)),
            pl.BlockSpec((T,), lambda j: (0,)),
        ],
        out_specs=pl.BlockSpec((T,), lambda j: (0,)),
        scratch_shapes=[
            pltpu.VMEM((2, T, BV), jnp.float32),
            pltpu.VMEM((2, T), jnp.float32),
            pltpu.VMEM((2, T), jnp.float32),
            pltpu.SemaphoreType.DMA,
            pltpu.SemaphoreType.DMA,
        ],
        compiler_params=pltpu.CompilerParams(
            dimension_semantics=("arbitrary",),
            collective_id=0,
            vmem_limit_bytes=100 * 1024 * 1024,
        ),
    )(x, W, labels)


# device time: 29374 ns/iter; 3.9465x vs baseline; 3.9465x over previous
import jax
import jax.numpy as jnp
from jax import lax
from jax.experimental import pallas as pl
from jax.experimental.pallas import tpu as pltpu

X_DIM, Y_DIM, Z_DIM = 2, 4, 4
N_DEV = X_DIM * Y_DIM * Z_DIM

OFFSETS = [
    (ox, oy, oz)
    for ox in range(X_DIM)
    for oy in range(Y_DIM)
    for oz in range(Z_DIM)
    if (ox, oy, oz) != (0, 0, 0)
]


def kernel(x, W, labels):
    T, D = x.shape
    _, V = W.shape
    BV = V * X_DIM // N_DEV

    def body(x_ref, w_hbm, l_ref, out_ref, wv_ref, acc_ref, recv_ref,
             w_sem, send_sems, recv_sems):
        my_x = lax.axis_index("x")
        my_y = lax.axis_index("y")
        my_z = lax.axis_index("z")
        sub = my_y * Z_DIM + my_z

        wdma = pltpu.make_async_copy(
            w_hbm.at[:, pl.ds(sub * BV, BV)], wv_ref, w_sem)
        wdma.start()
        wdma.wait()

        lg = jnp.dot(x_ref[:, :], wv_ref[:, :],
                     preferred_element_type=jnp.float32)
        bs = jnp.sum(jnp.exp(lg), axis=1)
        loc = l_ref[:] - (my_x * V + sub * BV)
        cols = lax.broadcasted_iota(jnp.int32, (T, BV), 1)
        lab = jnp.sum(jnp.where(cols == loc[:, None], lg, 0.0), axis=1)
        acc_ref[0, :] = bs
        acc_ref[1, :] = lab

        barrier = pltpu.get_barrier_semaphore()
        for (ox, oy, oz) in OFFSETS:
            tgt = ((my_x + ox) % X_DIM, (my_y + oy) % Y_DIM,
                   (my_z + oz) % Z_DIM)
            pl.semaphore_signal(barrier, inc=1, device_id=tgt,
                                device_id_type=pl.DeviceIdType.MESH)
        pl.semaphore_wait(barrier, len(OFFSETS))

        msgs = []
        for k, (ox, oy, oz) in enumerate(OFFSETS):
            tgt = ((my_x + ox) % X_DIM, (my_y + oy) % Y_DIM,
                   (my_z + oz) % Z_DIM)
            rdma = pltpu.make_async_remote_copy(
                src_ref=acc_ref,
                dst_ref=recv_ref.at[k],
                send_sem=send_sems.at[k],
                recv_sem=recv_sems.at[k],
                device_id=tgt,
                device_id_type=pl.DeviceIdType.MESH,
            )
            rdma.start()
            msgs.append(rdma)
        for m in msgs:
            m.wait_send()
        for m in msgs:
            m.wait_recv()

        tot = acc_ref[:, :] + jnp.sum(recv_ref[:, :, :], axis=0)
        out_ref[:] = jnp.log(tot[0, :]) - tot[1, :]

    return pl.pallas_call(
        body,
        out_shape=jax.ShapeDtypeStruct((T,), jnp.float32),
        in_specs=[
            pl.BlockSpec(memory_space=pltpu.VMEM),
            pl.BlockSpec(memory_space=pl.ANY),
            pl.BlockSpec(memory_space=pltpu.VMEM),
        ],
        out_specs=pl.BlockSpec(memory_space=pltpu.VMEM),
        scratch_shapes=[
            pltpu.VMEM((D, BV), jnp.float32),
            pltpu.VMEM((2, T), jnp.float32),
            pltpu.VMEM((len(OFFSETS), 2, T), jnp.float32),
            pltpu.SemaphoreType.DMA,
            pltpu.SemaphoreType.DMA((len(OFFSETS),)),
            pltpu.SemaphoreType.DMA((len(OFFSETS),)),
        ],
        compiler_params=pltpu.CompilerParams(
            collective_id=0,
            vmem_limit_bytes=100 * 1024 * 1024,
        ),
    )(x, W, labels)


# device time: 27736 ns/iter; 4.1796x vs baseline; 1.0591x over previous
import jax
import jax.numpy as jnp
from jax import lax
from jax.experimental import pallas as pl
from jax.experimental.pallas import tpu as pltpu

X_DIM, Y_DIM, Z_DIM = 2, 4, 4
N_DEV = X_DIM * Y_DIM * Z_DIM

OFFSETS = [
    (ox, oy, oz)
    for ox in range(X_DIM)
    for oy in range(Y_DIM)
    for oz in range(Z_DIM)
    if (ox, oy, oz) != (0, 0, 0)
]


def kernel(x, W, labels):
    T, D = x.shape
    _, V = W.shape
    BV = V * X_DIM // N_DEV

    def body(x_ref, w_hbm, l_ref, out_ref, wv_ref, acc_ref, recv_ref,
             w_sem, send_sems, recv_sems):
        my_x = lax.axis_index("x")
        my_y = lax.axis_index("y")
        my_z = lax.axis_index("z")
        sub = my_y * Z_DIM + my_z

        wdma = pltpu.make_async_copy(
            w_hbm.at[:, pl.ds(sub * BV, BV)], wv_ref, w_sem)
        wdma.start()

        barrier = pltpu.get_barrier_semaphore()
        for (ox, oy, oz) in OFFSETS:
            tgt = ((my_x + ox) % X_DIM, (my_y + oy) % Y_DIM,
                   (my_z + oz) % Z_DIM)
            pl.semaphore_signal(barrier, inc=1, device_id=tgt,
                                device_id_type=pl.DeviceIdType.MESH)

        wdma.wait()

        lg = jnp.dot(x_ref[:, :], wv_ref[:, :],
                     preferred_element_type=jnp.float32)
        bs = jnp.sum(jnp.exp(lg), axis=1)
        loc = l_ref[:] - (my_x * V + sub * BV)
        cols = lax.broadcasted_iota(jnp.int32, (T, BV), 1)
        lab = jnp.sum(jnp.where(cols == loc[:, None], lg, 0.0), axis=1)
        acc_ref[0, :] = bs
        acc_ref[1, :] = lab

        pl.semaphore_wait(barrier, len(OFFSETS))

        msgs = []
        for k, (ox, oy, oz) in enumerate(OFFSETS):
            tgt = ((my_x + ox) % X_DIM, (my_y + oy) % Y_DIM,
                   (my_z + oz) % Z_DIM)
            rdma = pltpu.make_async_remote_copy(
                src_ref=acc_ref,
                dst_ref=recv_ref.at[k],
                send_sem=send_sems.at[k],
                recv_sem=recv_sems.at[k],
                device_id=tgt,
                device_id_type=pl.DeviceIdType.MESH,
            )
            rdma.start()
            msgs.append(rdma)
        for m in msgs:
            m.wait_send()
        for m in msgs:
            m.wait_recv()

        tot = acc_ref[:, :] + jnp.sum(recv_ref[:, :, :], axis=0)
        out_ref[:] = jnp.log(tot[0, :]) - tot[1, :]

    return pl.pallas_call(
        body,
        out_shape=jax.ShapeDtypeStruct((T,), jnp.float32),
        in_specs=[
            pl.BlockSpec(memory_space=pltpu.VMEM),
            pl.BlockSpec(memory_space=pl.ANY),
            pl.BlockSpec(memory_space=pltpu.VMEM),
        ],
        out_specs=pl.BlockSpec(memory_space=pltpu.VMEM),
        scratch_shapes=[
            pltpu.VMEM((D, BV), jnp.float32),
            pltpu.VMEM((2, T), jnp.float32),
            pltpu.VMEM((len(OFFSETS), 2, T), jnp.float32),
            pltpu.SemaphoreType.DMA,
            pltpu.SemaphoreType.DMA((len(OFFSETS),)),
            pltpu.SemaphoreType.DMA((len(OFFSETS),)),
        ],
        compiler_params=pltpu.CompilerParams(
            collective_id=0,
            vmem_limit_bytes=100 * 1024 * 1024,
        ),
    )(x, W, labels)
